# baseline (speedup 1.0000x reference)
import functools

import jax
import jax.numpy as jnp
from jax import lax
from jax.experimental import pallas as pl
from jax.experimental.pallas import tpu as pltpu
from jax.experimental.pallas import tpu_sc as plsc

_CHUNK = 128
_NBUF = 2


@functools.cache
def _make_gather(batch, seq, V, D):
    B = batch * seq
    info = plsc.get_sparse_core_info()
    ns = info.num_subcores
    nw = info.num_cores * ns
    cols_per_w = batch // nw
    b_per_w = cols_per_w * seq
    cpr = cols_per_w // _CHUNK
    n_chunks = b_per_w // _CHUNK
    n_rings = n_chunks // _NBUF
    mesh = plsc.VectorSubcoreMesh(core_axis_name="c", subcore_axis_name="s")

    @functools.partial(
        pl.kernel,
        out_type=jax.ShapeDtypeStruct((B, D), jnp.float32),
        mesh=mesh,
        compiler_params=pltpu.CompilerParams(use_tc_tiling_on_sc=False),
        scratch_types=[
            pltpu.VMEM((seq, cols_per_w), jnp.int32),
            pltpu.VMEM((_NBUF, _CHUNK, D), jnp.float32),
            pltpu.VMEM_SHARED((ns, _NBUF, _CHUNK, D), jnp.float32),
            pltpu.SemaphoreType.DMA((_NBUF,)),
            pltpu.SemaphoreType.DMA((_NBUF,)),
            pltpu.SemaphoreType.DMA((_NBUF,)),
        ],
    )
    def gather_kernel(
        idx_hbm, table_hbm, out_hbm, idx_v, rows_v, rows_s, gsem, csem, wsem
    ):
        sid = lax.axis_index("s")
        wid = sid * info.num_cores + lax.axis_index("c")
        cbase = wid * cols_per_w
        pltpu.sync_copy(idx_hbm.at[:, pl.ds(cbase, cols_per_w)], idx_v)

        def out_slice(j):
            s, c = j // cpr, j % cpr
            return out_hbm.at[pl.ds(s * batch + cbase + c * _CHUNK, _CHUNK)]

        def idx_slice(j):
            s, c = j // cpr, j % cpr
            return idx_v.at[s, pl.ds(c * _CHUNK, _CHUNK)]

        def fire_gather(j, b):
            pltpu.async_copy(
                table_hbm.at[idx_slice(j)], rows_v.at[b], gsem.at[b]
            )

        def wait_gather(b):
            pltpu.make_async_copy(
                table_hbm.at[idx_slice(0)], rows_v.at[b], gsem.at[b]
            ).wait()

        def fire_xcopy(b):
            pltpu.async_copy(rows_v.at[b], rows_s.at[sid, b], csem.at[b])

        def wait_xcopy(b):
            pltpu.make_async_copy(
                rows_v.at[b], rows_s.at[sid, b], csem.at[b]
            ).wait()

        def fire_wb(j, b):
            pltpu.async_copy(rows_s.at[sid, b], out_slice(j), wsem.at[b])

        def wait_wb(j, b):
            pltpu.make_async_copy(
                rows_s.at[sid, b], out_slice(j), wsem.at[b]
            ).wait()

        for b in range(_NBUF):
            fire_gather(b, b)
        for b in range(_NBUF):
            wait_gather(b)
            fire_xcopy(b)
        for b in range(_NBUF):
            wait_xcopy(b)
            fire_gather(_NBUF + b, b)
            fire_wb(b, b)

        def ring_body(g, carry):
            jbase = g * _NBUF
            for b in range(_NBUF):
                wait_gather(b)
                wait_wb(jbase - _NBUF + b, b)
                fire_xcopy(b)
            for b in range(_NBUF):
                wait_xcopy(b)
                fire_gather(jbase + _NBUF + b, b)
                fire_wb(jbase + b, b)
            return carry

        lax.fori_loop(1, n_rings - 1, ring_body, 0)

        jbase = (n_rings - 1) * _NBUF
        for b in range(_NBUF):
            wait_gather(b)
            wait_wb(jbase - _NBUF + b, b)
            fire_xcopy(b)
        for b in range(_NBUF):
            wait_xcopy(b)
            fire_wb(jbase + b, b)
        for b in range(_NBUF):
            wait_wb(jbase + b, b)

    return gather_kernel


@jax.jit
def kernel(inputs, table):
    batch, seq = inputs.shape
    vocab, embed = table.shape
    out = _make_gather(batch, seq, vocab, embed)(inputs.T, table)
    return out.reshape(seq, batch, embed).transpose(1, 0, 2)


# speedup vs baseline: 1.0771x; 1.0771x over previous
import functools

import jax
import jax.numpy as jnp
from jax import lax
from jax.experimental import pallas as pl
from jax.experimental.pallas import tpu as pltpu
from jax.experimental.pallas import tpu_sc as plsc

_CHUNK = 256
_NBUF = 2


@functools.cache
def _make_gather(batch, seq, V, D):
    B = batch * seq
    info = plsc.get_sparse_core_info()
    nw = info.num_cores * info.num_subcores
    cols_per_w = batch // nw
    b_per_w = cols_per_w * seq
    cpr = cols_per_w // _CHUNK
    n_chunks = b_per_w // _CHUNK
    n_rings = n_chunks // _NBUF
    mesh = plsc.VectorSubcoreMesh(core_axis_name="c", subcore_axis_name="s")

    @functools.partial(
        pl.kernel,
        out_type=jax.ShapeDtypeStruct((B, D), jnp.float32),
        mesh=mesh,
        compiler_params=pltpu.CompilerParams(use_tc_tiling_on_sc=False),
        scratch_types=[
            pltpu.VMEM((seq, cols_per_w), jnp.int32),
            pltpu.VMEM((_NBUF, _CHUNK, D), jnp.float32),
            pltpu.SemaphoreType.DMA((_NBUF,)),
            pltpu.SemaphoreType.DMA((_NBUF,)),
        ],
    )
    def gather_kernel(idx_hbm, table_hbm, out_hbm, idx_v, rows_v, gsem, wsem):
        wid = lax.axis_index("s") * info.num_cores + lax.axis_index("c")
        cbase = wid * cols_per_w
        pltpu.sync_copy(idx_hbm.at[:, pl.ds(cbase, cols_per_w)], idx_v)

        def out_slice(j):
            s, c = j // cpr, j % cpr
            return out_hbm.at[pl.ds(s * batch + cbase + c * _CHUNK, _CHUNK)]

        def idx_slice(j):
            s, c = j // cpr, j % cpr
            return idx_v.at[s, pl.ds(c * _CHUNK, _CHUNK)]

        def fire(j, b):
            pltpu.async_copy(
                table_hbm.at[idx_slice(j)], rows_v.at[b], gsem.at[b]
            )

        def drain_fire_wb(j, b):
            pltpu.make_async_copy(
                table_hbm.at[idx_slice(0)], rows_v.at[b], gsem.at[b]
            ).wait()
            pltpu.async_copy(rows_v.at[b], out_slice(j), wsem.at[b])

        def wait_wb(j, b):
            pltpu.make_async_copy(rows_v.at[b], out_slice(j), wsem.at[b]).wait()

        for b in range(_NBUF):
            fire(b, b)

        def ring_body(g, carry):
            jbase = g * _NBUF
            for b in range(_NBUF):
                drain_fire_wb(jbase + b, b)
            for b in range(_NBUF):
                wait_wb(jbase + b, b)
                fire(jbase + _NBUF + b, b)
            return carry

        lax.fori_loop(0, n_rings - 1, ring_body, 0)

        jbase = (n_rings - 1) * _NBUF
        for b in range(_NBUF):
            drain_fire_wb(jbase + b, b)
        for b in range(_NBUF):
            wait_wb(jbase + b, b)

    return gather_kernel


@jax.jit
def kernel(inputs, table):
    batch, seq = inputs.shape
    vocab, embed = table.shape
    out = _make_gather(batch, seq, vocab, embed)(inputs.T, table)
    return out.reshape(seq, batch, embed).transpose(1, 0, 2)


# speedup vs baseline: 1.1206x; 1.0404x over previous
import functools

import jax
import jax.numpy as jnp
from jax import lax
from jax.experimental import pallas as pl
from jax.experimental.pallas import tpu as pltpu
from jax.experimental.pallas import tpu_sc as plsc

_CHUNK = 64
_NBUF = 8


@functools.cache
def _make_gather(batch, seq, V, D):
    B = batch * seq
    info = plsc.get_sparse_core_info()
    nw = info.num_cores * info.num_subcores
    cols_per_w = batch // nw
    b_per_w = cols_per_w * seq
    cpr = cols_per_w // _CHUNK
    n_chunks = b_per_w // _CHUNK
    n_rings = n_chunks // _NBUF
    mesh = plsc.VectorSubcoreMesh(core_axis_name="c", subcore_axis_name="s")

    @functools.partial(
        pl.kernel,
        out_type=jax.ShapeDtypeStruct((B, D), jnp.float32),
        mesh=mesh,
        compiler_params=pltpu.CompilerParams(use_tc_tiling_on_sc=False),
        scratch_types=[
            pltpu.VMEM((seq, cols_per_w), jnp.int32),
            pltpu.VMEM((_NBUF, _CHUNK, D), jnp.float32),
            pltpu.SemaphoreType.DMA((_NBUF,)),
            pltpu.SemaphoreType.DMA((_NBUF,)),
        ],
    )
    def gather_kernel(idx_hbm, table_hbm, out_hbm, idx_v, rows_v, gsem, wsem):
        wid = lax.axis_index("s") * info.num_cores + lax.axis_index("c")
        cbase = wid * cols_per_w
        pltpu.sync_copy(idx_hbm.at[:, pl.ds(cbase, cols_per_w)], idx_v)

        def out_slice(j):
            s, c = j // cpr, j % cpr
            return out_hbm.at[pl.ds(s * batch + cbase + c * _CHUNK, _CHUNK)]

        def idx_slice(j):
            s, c = j // cpr, j % cpr
            return idx_v.at[s, pl.ds(c * _CHUNK, _CHUNK)]

        def fire(j, b):
            pltpu.async_copy(
                table_hbm.at[idx_slice(j)], rows_v.at[b], gsem.at[b]
            )

        def drain_fire_wb(j, b):
            pltpu.make_async_copy(
                table_hbm.at[idx_slice(0)], rows_v.at[b], gsem.at[b]
            ).wait()
            pltpu.async_copy(rows_v.at[b], out_slice(j), wsem.at[b])

        def wait_wb(j, b):
            pltpu.make_async_copy(rows_v.at[b], out_slice(j), wsem.at[b]).wait()

        for b in range(_NBUF):
            fire(b, b)

        def ring_body(g, carry):
            jbase = g * _NBUF
            for b in range(_NBUF):
                drain_fire_wb(jbase + b, b)
            for b in range(_NBUF):
                wait_wb(jbase + b, b)
                fire(jbase + _NBUF + b, b)
            return carry

        lax.fori_loop(0, n_rings - 1, ring_body, 0)

        jbase = (n_rings - 1) * _NBUF
        for b in range(_NBUF):
            drain_fire_wb(jbase + b, b)
        for b in range(_NBUF):
            wait_wb(jbase + b, b)

    return gather_kernel


@jax.jit
def kernel(inputs, table):
    batch, seq = inputs.shape
    vocab, embed = table.shape
    out = _make_gather(batch, seq, vocab, embed)(inputs.T, table)
    return out.reshape(seq, batch, embed).transpose(1, 0, 2)
